# Initial kernel scaffold; baseline (speedup 1.0000x reference)
import jax
import jax.numpy as jnp
from jax.experimental import pallas as pl


def kernel(x, edge_index, W1_l, b1_l, W1_r, W2_l, b2_l, W2_r):
    raise NotImplementedError("write your pallas kernel here")


# speedup vs baseline: 5.6269x; 5.6269x over previous
import functools

import jax
import jax.numpy as jnp
from jax import lax
from jax.experimental import pallas as pl
from jax.experimental.pallas import tpu as pltpu
from jax.experimental.pallas import tpu_sc as plsc

_N = 10000
_E = 320000
_D = 128
_DP = 144
_NC = 2
_NS = 16
_NW = _NC * _NS
_EW = _E // _NW
_K = 80
_CH = _EW // _K
_RT = 632
_NP = _NS * _RT


@functools.cache
def _build_sc():
    mesh = plsc.VectorSubcoreMesh(core_axis_name="c", subcore_axis_name="s")
    return functools.partial(
        pl.kernel,
        out_type=jax.ShapeDtypeStruct((_NC, _NP, _DP), jnp.float32),
        mesh=mesh,
        scratch_types=[
            pltpu.VMEM_SHARED((_NP, _DP), jnp.float32),
            pltpu.VMEM((_K,), jnp.int32),
            pltpu.VMEM((_K,), jnp.int32),
            pltpu.VMEM((_K, _DP), jnp.float32),
            pltpu.SemaphoreType.DMA,
        ],
        compiler_params=pltpu.CompilerParams(use_tc_tiling_on_sc=False),
    )(_sc_scatter)


def _sc_scatter(xa, src, dst, zeros, acc_out, acc_sh, src_v, dst_v, rows_v, sem):
    cid = lax.axis_index("c")
    sid = lax.axis_index("s")
    wid = sid * _NC + cid
    r0 = sid * _RT

    pltpu.sync_copy(zeros.at[pl.ds(r0, _RT)], acc_sh.at[pl.ds(r0, _RT)])
    plsc.subcore_barrier()

    base = wid * _EW

    def body(i, carry):
        off = base + i * _K
        pltpu.sync_copy(src.at[pl.ds(off, _K)], src_v)
        pltpu.sync_copy(dst.at[pl.ds(off, _K)], dst_v)
        pltpu.async_copy(xa.at[src_v], rows_v, sem).wait()
        pltpu.sync_copy(rows_v, acc_sh.at[dst_v], add=True)
        return carry

    lax.fori_loop(0, _CH, body, 0)

    plsc.subcore_barrier()
    pltpu.sync_copy(acc_sh.at[pl.ds(r0, _RT)],
                    acc_out.at[cid, pl.ds(r0, _RT)])


_BN = 1000


def _tc_body(p_ref, x_ref, wl_ref, wr_ref, b_ref, o_ref):
    s = p_ref[0] + p_ref[1]
    cnt = jnp.maximum(s[:, _D:_D + 1], 1.0)
    mean = s[:, :_D] / cnt
    o_ref[...] = (
        jnp.dot(mean, wl_ref[...], preferred_element_type=jnp.float32)
        + jnp.dot(x_ref[...], wr_ref[...], preferred_element_type=jnp.float32)
        + b_ref[...]
    )


def _tc_combine(p, x, wl, wr, b):
    return pl.pallas_call(
        _tc_body,
        grid=(_N // _BN,),
        in_specs=[
            pl.BlockSpec((_NC, _BN, _DP), lambda i: (0, i, 0)),
            pl.BlockSpec((_BN, _D), lambda i: (i, 0)),
            pl.BlockSpec((_D, _D), lambda i: (0, 0)),
            pl.BlockSpec((_D, _D), lambda i: (0, 0)),
            pl.BlockSpec((1, _D), lambda i: (0, 0)),
        ],
        out_specs=pl.BlockSpec((_BN, _D), lambda i: (i, 0)),
        out_shape=jax.ShapeDtypeStruct((_N, _D), jnp.float32),
    )(p, x, wl, wr, b)


def kernel(x, edge_index, W1_l, b1_l, W1_r, W2_l, b2_l, W2_r):
    src = edge_index[0]
    dst = edge_index[1]
    xa = jnp.concatenate(
        [x, jnp.ones((_N, 1), x.dtype), jnp.zeros((_N, _DP - _D - 1), x.dtype)],
        axis=1)
    zeros = jnp.zeros((_NP, _DP), jnp.float32)
    acc = _build_sc()(xa, src, dst, zeros)
    return _tc_combine(acc, x, W2_l, W2_r, b2_l.reshape(1, _D))


# speedup vs baseline: 10.5623x; 1.8771x over previous
import functools

import jax
import jax.numpy as jnp
from jax import lax
from jax.experimental import pallas as pl
from jax.experimental.pallas import tpu as pltpu
from jax.experimental.pallas import tpu_sc as plsc

_N = 10000
_E = 320000
_D = 128
_DP = 136
_NC = 2
_NS = 16
_NW = _NC * _NS
_EW = _E // _NW
_K = 80
_CH = _EW // _K
_RT = 632
_NP = _NS * _RT


@functools.cache
def _build_sc():
    mesh = plsc.VectorSubcoreMesh(core_axis_name="c", subcore_axis_name="s")
    return functools.partial(
        pl.kernel,
        out_type=jax.ShapeDtypeStruct((_NC, _NP, _DP), jnp.float32),
        mesh=mesh,
        scratch_types=[
            pltpu.VMEM_SHARED((_NP, _DP), jnp.float32),
            pltpu.VMEM((_CH, _K), jnp.int32),
            pltpu.VMEM((_CH, _K), jnp.int32),
            pltpu.VMEM((_K, _DP), jnp.float32),
            pltpu.VMEM((_K, _DP), jnp.float32),
            pltpu.SemaphoreType.DMA,
            pltpu.SemaphoreType.DMA,
        ],
        compiler_params=pltpu.CompilerParams(use_tc_tiling_on_sc=False),
    )(_sc_scatter)


def _sc_scatter(xa, src3, dst3, zeros, acc_out,
                acc_sh, src_a, dst_a, buf0, buf1, sem0, sem1):
    cid = lax.axis_index("c")
    sid = lax.axis_index("s")
    wid = sid * _NC + cid
    r0 = sid * _RT

    pltpu.sync_copy(zeros.at[pl.ds(r0, _RT)], acc_sh.at[pl.ds(r0, _RT)])
    pltpu.sync_copy(src3.at[wid], src_a)
    pltpu.sync_copy(dst3.at[wid], dst_a)
    plsc.subcore_barrier()

    def start(i, buf, sem):
        pltpu.async_copy(xa.at[src_a.at[i]], buf, sem)

    def fin(i, buf, sem):
        pltpu.make_async_copy(xa.at[src_a.at[i]], buf, sem).wait()
        pltpu.sync_copy(buf, acc_sh.at[dst_a.at[i]], add=True)

    start(0, buf0, sem0)

    def body(j, carry):
        i = 2 * j
        start(i + 1, buf1, sem1)
        fin(i, buf0, sem0)
        start(i + 2, buf0, sem0)
        fin(i + 1, buf1, sem1)
        return carry

    lax.fori_loop(0, (_CH - 1) // 2, body, 0)
    fin(_CH - 1, buf0, sem0)

    plsc.subcore_barrier()
    pltpu.sync_copy(acc_sh.at[pl.ds(r0, _RT)],
                    acc_out.at[cid, pl.ds(r0, _RT)])


_BN = 1000


def _tc_body(p_ref, x_ref, wl_ref, wr_ref, b_ref, o_ref):
    s = p_ref[0] + p_ref[1]
    cnt = jnp.maximum(s[:, _D:_D + 1], 1.0)
    mean = s[:, :_D] / cnt
    o_ref[...] = (
        jnp.dot(mean, wl_ref[...], preferred_element_type=jnp.float32)
        + jnp.dot(x_ref[...], wr_ref[...], preferred_element_type=jnp.float32)
        + b_ref[...]
    )


def _tc_combine(p, x, wl, wr, b):
    return pl.pallas_call(
        _tc_body,
        grid=(_N // _BN,),
        in_specs=[
            pl.BlockSpec((_NC, _BN, _DP), lambda i: (0, i, 0)),
            pl.BlockSpec((_BN, _D), lambda i: (i, 0)),
            pl.BlockSpec((_D, _D), lambda i: (0, 0)),
            pl.BlockSpec((_D, _D), lambda i: (0, 0)),
            pl.BlockSpec((1, _D), lambda i: (0, 0)),
        ],
        out_specs=pl.BlockSpec((_BN, _D), lambda i: (i, 0)),
        out_shape=jax.ShapeDtypeStruct((_N, _D), jnp.float32),
    )(p, x, wl, wr, b)


def kernel(x, edge_index, W1_l, b1_l, W1_r, W2_l, b2_l, W2_r):
    src = edge_index[0].reshape(_NW, _CH, _K)
    dst = edge_index[1].reshape(_NW, _CH, _K)
    xa = jnp.concatenate(
        [x, jnp.ones((_N, 1), x.dtype), jnp.zeros((_N, _DP - _D - 1), x.dtype)],
        axis=1)
    zeros = jnp.zeros((_NP, _DP), jnp.float32)
    acc = _build_sc()(xa, src, dst, zeros)
    return _tc_combine(acc, x, W2_l, W2_r, b2_l.reshape(1, _D))
